# Initial kernel scaffold; baseline (speedup 1.0000x reference)
import jax
import jax.numpy as jnp
from jax.experimental import pallas as pl


def kernel(x0, edge_index0, x1, edge_index1, W1_0, b1_0, W2_0, b2_0, W1_1, b1_1, W2_1, b2_1):
    raise NotImplementedError("write your pallas kernel here")


# speedup vs baseline: 12.0528x; 12.0528x over previous
import functools

import jax
import jax.numpy as jnp
from jax import lax
from jax.experimental import pallas as pl
from jax.experimental.pallas import tpu as pltpu
from jax.experimental.pallas import tpu_sc as plsc

N = 10000
E = 320000
D_IN = 128
D_HID = 64
D_OUT = 128

NT = 16
CHUNK = 128
IB = 16
NB = 10
CPT = NB * IB
EPT = CPT * CHUNK
EPAD = EPT * NT
RPT = 640
NPAD = RPT * NT
DEGW = 16
RB = 1280
GRID_R = NPAD // RB


def _make_deg_kernel():
    mesh = plsc.VectorSubcoreMesh(core_axis_name="c", subcore_axis_name="s")

    @functools.partial(
        pl.kernel,
        mesh=mesh,
        compiler_params=pltpu.CompilerParams(use_tc_tiling_on_sc=False),
        out_type=jax.ShapeDtypeStruct((2, NPAD, DEGW), jnp.float32),
        scratch_types=[
            pltpu.VMEM((IB, CHUNK), jnp.int32),
            pltpu.VMEM((CHUNK, DEGW), jnp.float32),
            pltpu.VMEM_SHARED((NPAD, DEGW), jnp.float32),
        ],
    )
    def deg_kernel(dst_hbm, ones_hbm, zeros_hbm, out_hbm, dst_v, ones_v, acc):
        g = lax.axis_index("c")
        s = lax.axis_index("s")
        base = s * RPT
        pltpu.sync_copy(ones_hbm, ones_v)
        pltpu.sync_copy(zeros_hbm, acc.at[pl.ds(base, RPT)])
        plsc.subcore_barrier()

        def block(b, carry):
            pltpu.sync_copy(dst_hbm.at[g, s, b], dst_v)

            def step(c, carry2):
                pltpu.sync_copy(ones_v, acc.at[dst_v.at[c]], add=True)
                return carry2

            lax.fori_loop(0, IB, step, 0)
            return carry

        lax.fori_loop(0, NB, block, 0)
        plsc.subcore_barrier()
        pltpu.sync_copy(acc.at[pl.ds(base, RPT)], out_hbm.at[g, pl.ds(base, RPT)])

    return deg_kernel


def _make_gs_kernel(d):
    mesh = plsc.VectorSubcoreMesh(core_axis_name="c", subcore_axis_name="s")

    @functools.partial(
        pl.kernel,
        mesh=mesh,
        compiler_params=pltpu.CompilerParams(use_tc_tiling_on_sc=False),
        out_type=jax.ShapeDtypeStruct((2, NPAD, d), jnp.float32),
        scratch_types=[
            pltpu.VMEM((IB, CHUNK), jnp.int32),
            pltpu.VMEM((IB, CHUNK), jnp.int32),
            pltpu.VMEM((CHUNK, d), jnp.float32),
            pltpu.VMEM_SHARED((NPAD, d), jnp.float32),
            pltpu.SemaphoreType.DMA,
        ],
    )
    def gs_kernel(src_hbm, dst_hbm, y_hbm, out_hbm, src_v, dst_v, rows_v, acc, sem):
        g = lax.axis_index("c")
        s = lax.axis_index("s")
        base = s * RPT
        pltpu.sync_copy(y_hbm.at[pl.ds(g * NPAD + base, RPT)], acc.at[pl.ds(base, RPT)])
        plsc.subcore_barrier()

        def block(b, carry):
            pltpu.sync_copy(src_hbm.at[g, s, b], src_v)
            pltpu.sync_copy(dst_hbm.at[g, s, b], dst_v)

            def step(c, carry2):
                pltpu.async_copy(y_hbm.at[src_v.at[c]], rows_v, sem).wait()
                pltpu.sync_copy(rows_v, acc.at[dst_v.at[c]], add=True)
                return carry2

            lax.fori_loop(0, IB, step, 0)
            return carry

        lax.fori_loop(0, NB, block, 0)
        plsc.subcore_barrier()
        pltpu.sync_copy(acc.at[pl.ds(base, RPT)], out_hbm.at[g, pl.ds(base, RPT)])

    return gs_kernel


_deg_call = _make_deg_kernel()
_gs64 = _make_gs_kernel(D_HID)
_gs128 = _make_gs_kernel(D_OUT)


def _tc_pre(x_ref, deg_ref, w_ref, y_ref):
    dinv = lax.rsqrt(deg_ref[0][:, 0:1] + 1.0)
    y_ref[0] = dinv * jnp.dot(x_ref[0], w_ref[0], preferred_element_type=jnp.float32)


def _tc_mid(s_ref, deg_ref, b_ref, w_ref, y2_ref):
    dinv = lax.rsqrt(deg_ref[0][:, 0:1] + 1.0)
    o = dinv * s_ref[0] + b_ref[0, 0]
    h = jnp.where(o > 0, o, jnp.exp(o) - 1.0)
    y2_ref[0] = dinv * jnp.dot(h, w_ref[0], preferred_element_type=jnp.float32)


def _tc_post(s_ref, deg_ref, b_ref, h_ref):
    dinv = lax.rsqrt(deg_ref[0][:, 0:1] + 1.0)
    o = dinv * s_ref[0] + b_ref[0, 0]
    h_ref[0] = jnp.where(o > 0, o, jnp.exp(o) - 1.0)


def kernel(x0, edge_index0, x1, edge_index1, W1_0, b1_0, W2_0, b2_0, W1_1, b1_1, W2_1, b2_1):
    f32 = jnp.float32
    xs = jnp.stack([
        jnp.pad(x0, ((0, NPAD - N), (0, 0))),
        jnp.pad(x1, ((0, NPAD - N), (0, 0))),
    ])
    pad = jnp.full((EPAD - E,), N, dtype=jnp.int32)
    src = jnp.stack([
        jnp.concatenate([edge_index0[0], pad]),
        jnp.concatenate([edge_index1[0], pad]) + NPAD,
    ]).reshape(2, NT, NB, IB, CHUNK)
    dst = jnp.stack([
        jnp.concatenate([edge_index0[1], pad]),
        jnp.concatenate([edge_index1[1], pad]),
    ]).reshape(2, NT, NB, IB, CHUNK)
    ones = jnp.ones((CHUNK, DEGW), f32)
    zeros = jnp.zeros((RPT, DEGW), f32)

    deg16 = _deg_call(dst, ones, zeros)

    w1 = jnp.stack([W1_0, W1_1])
    w2 = jnp.stack([W2_0, W2_1])
    b1 = jnp.stack([b1_0, b1_1]).reshape(2, 1, D_HID)
    b2 = jnp.stack([b2_0, b2_1]).reshape(2, 1, D_OUT)

    y1 = pl.pallas_call(
        _tc_pre,
        grid=(2, GRID_R),
        in_specs=[
            pl.BlockSpec((1, RB, D_IN), lambda g, i: (g, i, 0)),
            pl.BlockSpec((1, RB, DEGW), lambda g, i: (g, i, 0)),
            pl.BlockSpec((1, D_IN, D_HID), lambda g, i: (g, 0, 0)),
        ],
        out_specs=pl.BlockSpec((1, RB, D_HID), lambda g, i: (g, i, 0)),
        out_shape=jax.ShapeDtypeStruct((2, NPAD, D_HID), f32),
    )(xs, deg16, w1)

    s1 = _gs64(src, dst, y1.reshape(2 * NPAD, D_HID))

    y2 = pl.pallas_call(
        _tc_mid,
        grid=(2, GRID_R),
        in_specs=[
            pl.BlockSpec((1, RB, D_HID), lambda g, i: (g, i, 0)),
            pl.BlockSpec((1, RB, DEGW), lambda g, i: (g, i, 0)),
            pl.BlockSpec((1, 1, D_HID), lambda g, i: (g, 0, 0)),
            pl.BlockSpec((1, D_HID, D_OUT), lambda g, i: (g, 0, 0)),
        ],
        out_specs=pl.BlockSpec((1, RB, D_OUT), lambda g, i: (g, i, 0)),
        out_shape=jax.ShapeDtypeStruct((2, NPAD, D_OUT), f32),
    )(s1, deg16, b1, w2)

    s2 = _gs128(src, dst, y2.reshape(2 * NPAD, D_OUT))

    h2 = pl.pallas_call(
        _tc_post,
        grid=(2, GRID_R),
        in_specs=[
            pl.BlockSpec((1, RB, D_OUT), lambda g, i: (g, i, 0)),
            pl.BlockSpec((1, RB, DEGW), lambda g, i: (g, i, 0)),
            pl.BlockSpec((1, 1, D_OUT), lambda g, i: (g, 0, 0)),
        ],
        out_specs=pl.BlockSpec((1, RB, D_OUT), lambda g, i: (g, i, 0)),
        out_shape=jax.ShapeDtypeStruct((2, NPAD, D_OUT), f32),
    )(s2, deg16, b2)

    return h2[:, :N, :].reshape(2 * N, D_OUT)


# speedup vs baseline: 14.9830x; 1.2431x over previous
import functools

import jax
import jax.numpy as jnp
from jax import lax
from jax.experimental import pallas as pl
from jax.experimental.pallas import tpu as pltpu
from jax.experimental.pallas import tpu_sc as plsc

N = 10000
E = 320000
D_IN = 128
D_HID = 64
D_OUT = 128

NT = 16
CHUNK = 128
IB = 16
NB = 10
CPT = NB * IB
EPT = CPT * CHUNK
EPAD = EPT * NT
RPT = 640
NPAD = RPT * NT
DEGW = 16
RB = 1280
GRID_R = NPAD // RB


def _make_deg_kernel():
    mesh = plsc.VectorSubcoreMesh(core_axis_name="c", subcore_axis_name="s")

    @functools.partial(
        pl.kernel,
        mesh=mesh,
        compiler_params=pltpu.CompilerParams(use_tc_tiling_on_sc=False),
        out_type=jax.ShapeDtypeStruct((2, NPAD, DEGW), jnp.float32),
        scratch_types=[
            pltpu.VMEM((IB, CHUNK), jnp.int32),
            pltpu.VMEM((CHUNK, DEGW), jnp.float32),
            pltpu.VMEM_SHARED((NPAD, DEGW), jnp.float32),
        ],
    )
    def deg_kernel(dst_hbm, ones_hbm, zeros_hbm, out_hbm, dst_v, ones_v, acc):
        g = lax.axis_index("c")
        s = lax.axis_index("s")
        base = s * RPT
        pltpu.sync_copy(ones_hbm, ones_v)
        pltpu.sync_copy(zeros_hbm, acc.at[pl.ds(base, RPT)])
        plsc.subcore_barrier()

        def block(b, carry):
            pltpu.sync_copy(dst_hbm.at[g, s, b], dst_v)

            def step(c, carry2):
                pltpu.sync_copy(ones_v, acc.at[dst_v.at[c]], add=True)
                return carry2

            lax.fori_loop(0, IB, step, 0)
            return carry

        lax.fori_loop(0, NB, block, 0)
        plsc.subcore_barrier()
        pltpu.sync_copy(acc.at[pl.ds(base, RPT)], out_hbm.at[g, pl.ds(base, RPT)])

    return deg_kernel


def _make_gs_kernel(d):
    mesh = plsc.VectorSubcoreMesh(core_axis_name="c", subcore_axis_name="s")

    @functools.partial(
        pl.kernel,
        mesh=mesh,
        compiler_params=pltpu.CompilerParams(use_tc_tiling_on_sc=False),
        out_type=jax.ShapeDtypeStruct((2, NPAD, d), jnp.float32),
        scratch_types=[
            pltpu.VMEM((2, IB, CHUNK), jnp.int32),
            pltpu.VMEM((2, IB, CHUNK), jnp.int32),
            pltpu.VMEM((2, CHUNK, d), jnp.float32),
            pltpu.VMEM_SHARED((NPAD, d), jnp.float32),
            pltpu.SemaphoreType.DMA,
            pltpu.SemaphoreType.DMA,
        ],
    )
    def gs_kernel(src_hbm, dst_hbm, y_hbm, out_hbm, src_v, dst_v, rows_v, acc, isem, gsem):
        g = lax.axis_index("c")
        s = lax.axis_index("s")
        base = s * RPT
        pltpu.async_copy(src_hbm.at[g, s, 0], src_v.at[0], isem)
        pltpu.async_copy(dst_hbm.at[g, s, 0], dst_v.at[0], isem)
        pltpu.sync_copy(y_hbm.at[pl.ds(g * NPAD + base, RPT)], acc.at[pl.ds(base, RPT)])
        pltpu.make_async_copy(src_hbm.at[g, s, 0], src_v.at[0], isem).wait()
        pltpu.make_async_copy(dst_hbm.at[g, s, 0], dst_v.at[0], isem).wait()
        plsc.subcore_barrier()
        pltpu.async_copy(y_hbm.at[src_v.at[0, 0]], rows_v.at[0], gsem)

        def block(b, carry):
            bb = b % 2
            nb = (b + 1) % 2

            @pl.when(b + 1 < NB)
            def _():
                pltpu.async_copy(src_hbm.at[g, s, b + 1], src_v.at[nb], isem)
                pltpu.async_copy(dst_hbm.at[g, s, b + 1], dst_v.at[nb], isem)

            def step(c, carry2):
                cur = c % 2
                nxt = (c + 1) % 2

                @pl.when(c + 1 < IB)
                def _():
                    pltpu.async_copy(y_hbm.at[src_v.at[bb, c + 1]], rows_v.at[nxt], gsem)

                @pl.when(jnp.logical_and(c + 1 == IB, b + 1 < NB))
                def _():
                    pltpu.make_async_copy(src_hbm.at[g, s, b + 1], src_v.at[nb], isem).wait()
                    pltpu.make_async_copy(dst_hbm.at[g, s, b + 1], dst_v.at[nb], isem).wait()
                    pltpu.async_copy(y_hbm.at[src_v.at[nb, 0]], rows_v.at[nxt], gsem)

                pltpu.make_async_copy(y_hbm.at[src_v.at[bb, c]], rows_v.at[cur], gsem).wait()
                pltpu.sync_copy(rows_v.at[cur], acc.at[dst_v.at[bb, c]], add=True)
                return carry2

            lax.fori_loop(0, IB, step, 0)
            return carry

        lax.fori_loop(0, NB, block, 0)
        plsc.subcore_barrier()
        pltpu.sync_copy(acc.at[pl.ds(base, RPT)], out_hbm.at[g, pl.ds(base, RPT)])

    return gs_kernel


_deg_call = _make_deg_kernel()
_gs64 = _make_gs_kernel(D_HID)
_gs128 = _make_gs_kernel(D_OUT)


def _tc_pre(x_ref, deg_ref, w_ref, y_ref):
    dinv = lax.rsqrt(deg_ref[0][:, 0:1] + 1.0)
    y_ref[0] = dinv * jnp.dot(x_ref[0], w_ref[0], preferred_element_type=jnp.float32)


def _tc_mid(s_ref, deg_ref, b_ref, w_ref, y2_ref):
    dinv = lax.rsqrt(deg_ref[0][:, 0:1] + 1.0)
    o = dinv * s_ref[0] + b_ref[0, 0]
    h = jnp.where(o > 0, o, jnp.exp(o) - 1.0)
    y2_ref[0] = dinv * jnp.dot(h, w_ref[0], preferred_element_type=jnp.float32)


def _tc_post(s_ref, deg_ref, b_ref, h_ref):
    dinv = lax.rsqrt(deg_ref[0][:, 0:1] + 1.0)
    o = dinv * s_ref[0] + b_ref[0, 0]
    h_ref[0] = jnp.where(o > 0, o, jnp.exp(o) - 1.0)


def kernel(x0, edge_index0, x1, edge_index1, W1_0, b1_0, W2_0, b2_0, W1_1, b1_1, W2_1, b2_1):
    f32 = jnp.float32
    xs = jnp.stack([
        jnp.pad(x0, ((0, NPAD - N), (0, 0))),
        jnp.pad(x1, ((0, NPAD - N), (0, 0))),
    ])
    pad = jnp.full((EPAD - E,), N, dtype=jnp.int32)
    src = jnp.stack([
        jnp.concatenate([edge_index0[0], pad]),
        jnp.concatenate([edge_index1[0], pad]) + NPAD,
    ]).reshape(2, NT, NB, IB, CHUNK)
    dst = jnp.stack([
        jnp.concatenate([edge_index0[1], pad]),
        jnp.concatenate([edge_index1[1], pad]),
    ]).reshape(2, NT, NB, IB, CHUNK)
    ones = jnp.ones((CHUNK, DEGW), f32)
    zeros = jnp.zeros((RPT, DEGW), f32)

    deg16 = _deg_call(dst, ones, zeros)

    w1 = jnp.stack([W1_0, W1_1])
    w2 = jnp.stack([W2_0, W2_1])
    b1 = jnp.stack([b1_0, b1_1]).reshape(2, 1, D_HID)
    b2 = jnp.stack([b2_0, b2_1]).reshape(2, 1, D_OUT)

    y1 = pl.pallas_call(
        _tc_pre,
        grid=(2, GRID_R),
        in_specs=[
            pl.BlockSpec((1, RB, D_IN), lambda g, i: (g, i, 0)),
            pl.BlockSpec((1, RB, DEGW), lambda g, i: (g, i, 0)),
            pl.BlockSpec((1, D_IN, D_HID), lambda g, i: (g, 0, 0)),
        ],
        out_specs=pl.BlockSpec((1, RB, D_HID), lambda g, i: (g, i, 0)),
        out_shape=jax.ShapeDtypeStruct((2, NPAD, D_HID), f32),
    )(xs, deg16, w1)

    s1 = _gs64(src, dst, y1.reshape(2 * NPAD, D_HID))

    y2 = pl.pallas_call(
        _tc_mid,
        grid=(2, GRID_R),
        in_specs=[
            pl.BlockSpec((1, RB, D_HID), lambda g, i: (g, i, 0)),
            pl.BlockSpec((1, RB, DEGW), lambda g, i: (g, i, 0)),
            pl.BlockSpec((1, 1, D_HID), lambda g, i: (g, 0, 0)),
            pl.BlockSpec((1, D_HID, D_OUT), lambda g, i: (g, 0, 0)),
        ],
        out_specs=pl.BlockSpec((1, RB, D_OUT), lambda g, i: (g, i, 0)),
        out_shape=jax.ShapeDtypeStruct((2, NPAD, D_OUT), f32),
    )(s1, deg16, b1, w2)

    s2 = _gs128(src, dst, y2.reshape(2 * NPAD, D_OUT))

    h2 = pl.pallas_call(
        _tc_post,
        grid=(2, GRID_R),
        in_specs=[
            pl.BlockSpec((1, RB, D_OUT), lambda g, i: (g, i, 0)),
            pl.BlockSpec((1, RB, DEGW), lambda g, i: (g, i, 0)),
            pl.BlockSpec((1, 1, D_OUT), lambda g, i: (g, 0, 0)),
        ],
        out_specs=pl.BlockSpec((1, RB, D_OUT), lambda g, i: (g, i, 0)),
        out_shape=jax.ShapeDtypeStruct((2, NPAD, D_OUT), f32),
    )(s2, deg16, b2)

    return h2[:, :N, :].reshape(2 * N, D_OUT)
